# baseline (speedup 1.0000x reference)
import functools

import jax
import jax.numpy as jnp
from jax import lax
from jax.experimental import pallas as pl
from jax.experimental.pallas import tpu as pltpu
from jax.experimental.pallas import tpu_sc as plsc

_NUM_CLASSES = 1000000
_FEAT = 64
_BATCH = 16384
_LAMBDA = 0.003

_INFO = plsc.get_sparse_core_info()
_NC, _NS, _L = _INFO.num_cores, _INFO.num_subcores, _INFO.num_lanes
_NW = _NC * _NS
_BPW = _BATCH // _NW
_CHUNK = 128
_NCHUNK = _BPW // _CHUNK
_RB = _BPW // _L
_EROW = _BPW * _FEAT // 128


def _body(labels_hbm, emb_hbm, centers_hbm, out_hbm,
          lab_v, idx_v, par_v, rows_v, emb_v, out_v, sem_g, sem_e):
    wid = lax.axis_index("s") * _NC + lax.axis_index("c")
    base = wid * _BPW

    pltpu.sync_copy(labels_hbm.at[pl.ds(base, _BPW)], lab_v)
    emb_cp = pltpu.async_copy(
        emb_hbm.at[pl.ds(wid * _EROW, _EROW), :], emb_v, sem_e)

    def prep(i, carry):
        lv = lab_v[pl.ds(i * _L, _L)]
        idx_v[pl.ds(i * _L, _L)] = lax.shift_right_logical(lv, 1)
        par_v[pl.ds(i * _L, _L)] = lax.shift_left(
            lax.bitwise_and(lv, 1), 6)
        return carry

    lax.fori_loop(0, _RB, prep, 0)

    gathers = [
        pltpu.async_copy(
            centers_hbm.at[idx_v.at[pl.ds(j * _CHUNK, _CHUNK)]],
            rows_v.at[pl.ds(j * _CHUNK, _CHUNK), :],
            sem_g,
        )
        for j in range(_NCHUNK)
    ]
    for g in gathers:
        g.wait()
    emb_cp.wait()

    iota = lax.iota(jnp.int32, _L)
    iotad2 = lax.shift_right_logical(iota, 1)
    ehalf = lax.shift_left(lax.bitwise_and(iota, 1), 6)

    def step(f, accs):
        fvec = jnp.full((_L,), f, jnp.int32)
        evec1 = ehalf + fvec
        out = list(accs)
        for rb in range(_RB):
            i0 = iota + (rb * _L)
            e0 = iotad2 + (rb * (_L // 2))
            p64 = par_v[pl.ds(rb * _L, _L)]
            cv = plsc.load_gather(rows_v, [i0, p64 + fvec])
            ev = plsc.load_gather(emb_v, [e0, evec1])
            d = ev - cv
            out[rb % 4] = out[rb % 4] + d * d
        return tuple(out)

    zero = jnp.zeros((_L,), jnp.float32)
    accs = lax.fori_loop(0, _FEAT, step, (zero,) * 4)
    out_v[...] = (accs[0] + accs[1]) + (accs[2] + accs[3])
    pltpu.sync_copy(out_v, out_hbm.at[pl.ds(wid * _L, _L)])


@jax.jit
def _center_loss_partials(labels, emb128, centers2):
    mesh = plsc.VectorSubcoreMesh(core_axis_name="c", subcore_axis_name="s")
    k = functools.partial(
        pl.kernel,
        mesh=mesh,
        out_type=jax.ShapeDtypeStruct((_NW * _L,), jnp.float32),
        scratch_types=[
            pltpu.VMEM((_BPW,), jnp.int32),
            pltpu.VMEM((_BPW,), jnp.int32),
            pltpu.VMEM((_BPW,), jnp.int32),
            pltpu.VMEM((_BPW, 128), jnp.float32),
            pltpu.VMEM((_EROW, 128), jnp.float32),
            pltpu.VMEM((_L,), jnp.float32),
            pltpu.SemaphoreType.DMA,
            pltpu.SemaphoreType.DMA,
        ],
        compiler_params=pltpu.CompilerParams(
            use_tc_tiling_on_sc=True, needs_layout_passes=False),
    )(_body)
    return k(labels, emb128, centers2)


def kernel(embeddings, labels, centers):
    emb128 = embeddings.reshape(_BATCH * _FEAT // 128, 128)
    centers2 = centers.reshape(_NUM_CLASSES // 2, 128)
    partials = _center_loss_partials(
        labels.astype(jnp.int32), emb128, centers2)
    return jnp.sum(partials) * (_LAMBDA / (_BATCH * _FEAT))


# speedup vs baseline: 1.1918x; 1.1918x over previous
import functools

import jax
import jax.numpy as jnp
from jax import lax
from jax.experimental import pallas as pl
from jax.experimental.pallas import tpu as pltpu
from jax.experimental.pallas import tpu_sc as plsc

_NUM_CLASSES = 1000000
_FEAT = 64
_BATCH = 16384
_LAMBDA = 0.003

_INFO = plsc.get_sparse_core_info()
_NC, _NS, _L = _INFO.num_cores, _INFO.num_subcores, _INFO.num_lanes
_NW = _NC * _NS
_BPW = _BATCH // _NW
_CHUNK = 128
_NCHUNK = _BPW // _CHUNK
_FVEC = _FEAT // _L
_EROW = _BPW * _FEAT // 128


def _body(labels_hbm, emb_hbm, centers_hbm, out_hbm,
          idx_v, rows_v, emb_v, out_v, sem_g, sem_e):
    wid = lax.axis_index("s") * _NC + lax.axis_index("c")
    base = wid * _BPW

    pltpu.sync_copy(labels_hbm.at[pl.ds(base, _BPW)], idx_v)
    emb_cp = pltpu.async_copy(
        emb_hbm.at[pl.ds(wid * _EROW, _EROW), :], emb_v, sem_e)
    gathers = [
        pltpu.async_copy(
            centers_hbm.at[idx_v.at[pl.ds(j * _CHUNK, _CHUNK)]],
            rows_v.at[pl.ds(j * _CHUNK, _CHUNK), :],
            sem_g,
        )
        for j in range(_NCHUNK)
    ]
    for g in gathers:
        g.wait()
    emb_cp.wait()

    def step(k, accs):
        out = list(accs)
        for half in range(2):
            r = 2 * k + half
            for c in range(_FVEC):
                ev = emb_v[k, pl.ds(half * _FEAT + c * _L, _L)]
                cv = rows_v[r, pl.ds(c * _L, _L)]
                d = ev - cv
                out[c] = out[c] + d * d
        return tuple(out)

    zero = jnp.zeros((_L,), jnp.float32)
    accs = lax.fori_loop(0, _BPW // 2, step, (zero,) * _FVEC)
    out_v[...] = (accs[0] + accs[1]) + (accs[2] + accs[3])
    pltpu.sync_copy(out_v, out_hbm.at[pl.ds(wid * _L, _L)])


@jax.jit
def _center_loss_partials(labels, emb128, centers_pad):
    mesh = plsc.VectorSubcoreMesh(core_axis_name="c", subcore_axis_name="s")
    k = functools.partial(
        pl.kernel,
        mesh=mesh,
        out_type=jax.ShapeDtypeStruct((_NW * _L,), jnp.float32),
        scratch_types=[
            pltpu.VMEM((_BPW,), jnp.int32),
            pltpu.VMEM((_BPW, 128), jnp.float32),
            pltpu.VMEM((_EROW, 128), jnp.float32),
            pltpu.VMEM((_L,), jnp.float32),
            pltpu.SemaphoreType.DMA,
            pltpu.SemaphoreType.DMA,
        ],
        compiler_params=pltpu.CompilerParams(
            use_tc_tiling_on_sc=True, needs_layout_passes=False),
    )(_body)
    return k(labels, emb128, centers_pad)


def kernel(embeddings, labels, centers):
    emb128 = embeddings.reshape(_BATCH * _FEAT // 128, 128)
    centers_pad = jnp.pad(centers, ((0, 0), (0, 128 - _FEAT)))
    partials = _center_loss_partials(
        labels.astype(jnp.int32), emb128, centers_pad)
    return jnp.sum(partials) * (_LAMBDA / (_BATCH * _FEAT))


# speedup vs baseline: 1.6928x; 1.4204x over previous
import functools

import jax
import jax.numpy as jnp
from jax import lax
from jax.experimental import pallas as pl
from jax.experimental.pallas import tpu as pltpu
from jax.experimental.pallas import tpu_sc as plsc

_NUM_CLASSES = 1000000
_FEAT = 64
_BATCH = 16384
_LAMBDA = 0.003

_INFO = plsc.get_sparse_core_info()
_NC, _NS, _L = _INFO.num_cores, _INFO.num_subcores, _INFO.num_lanes
_NW = _NC * _NS
_BPW = _BATCH // _NW
_NG = _BPW // _L
_NBUF = 4
_AHEAD = 3
_FVEC = _FEAT // _L
_EROW = _BPW * _FEAT // 128


def _body(labels_hbm, emb_hbm, centers_hbm, out_hbm,
          lab_v, blocks_v, emb_v, out_v, sem_b, sem_e):
    wid = lax.axis_index("s") * _NC + lax.axis_index("c")
    base = wid * _BPW

    pltpu.sync_copy(labels_hbm.at[pl.ds(base, _BPW)], lab_v)
    emb_cp = pltpu.async_copy(
        emb_hbm.at[pl.ds(wid * _EROW, _EROW), :], emb_v, sem_e)

    def fire(g):
        lv = lab_v[pl.ds(g * _L, _L)]
        av = lax.shift_left(lax.shift_right_logical(lv, 3), 3)
        s = lax.rem(g, _NBUF)
        for b in range(_L):
            al = pl.multiple_of(av[b], 8)
            pltpu.async_copy(
                centers_hbm.at[pl.ds(al, 8), :],
                blocks_v.at[s, b], sem_b)

    for g in range(_AHEAD):
        fire(g)
    emb_cp.wait()

    def step(g, accs):
        @pl.when(g < _NG - _AHEAD)
        def _():
            fire(g + _AHEAD)
        s = lax.rem(g, _NBUF)
        for b in range(_L):
            pltpu.make_async_copy(
                centers_hbm.at[pl.ds(0, 8), :],
                blocks_v.at[s, b], sem_b).wait()
        lv = lab_v[pl.ds(g * _L, _L)]
        ov = lax.bitwise_and(lv, 7)
        out = list(accs)
        for b in range(_L):
            ob = ov[b]
            k = g * (_L // 2) + (b // 2)
            for c in range(_FVEC):
                ev = emb_v[k, pl.ds((b % 2) * _FEAT + c * _L, _L)]
                cv = blocks_v[s, b, ob, pl.ds(c * _L, _L)]
                d = ev - cv
                out[c] = out[c] + d * d
        return tuple(out)

    zero = jnp.zeros((_L,), jnp.float32)
    accs = lax.fori_loop(0, _NG, step, (zero,) * _FVEC)
    out_v[...] = (accs[0] + accs[1]) + (accs[2] + accs[3])
    pltpu.sync_copy(out_v, out_hbm.at[pl.ds(wid * _L, _L)])


@jax.jit
def _center_loss_partials(labels, emb128, centers):
    mesh = plsc.VectorSubcoreMesh(core_axis_name="c", subcore_axis_name="s")
    k = functools.partial(
        pl.kernel,
        mesh=mesh,
        out_type=jax.ShapeDtypeStruct((_NW * _L,), jnp.float32),
        scratch_types=[
            pltpu.VMEM((_BPW,), jnp.int32),
            pltpu.VMEM((_NBUF, _L, 8, _FEAT), jnp.float32),
            pltpu.VMEM((_EROW, 128), jnp.float32),
            pltpu.VMEM((_L,), jnp.float32),
            pltpu.SemaphoreType.DMA,
            pltpu.SemaphoreType.DMA,
        ],
        compiler_params=pltpu.CompilerParams(
            use_tc_tiling_on_sc=True, needs_layout_passes=False),
    )(_body)
    return k(labels, emb128, centers)


def kernel(embeddings, labels, centers):
    emb128 = embeddings.reshape(_BATCH * _FEAT // 128, 128)
    partials = _center_loss_partials(
        labels.astype(jnp.int32), emb128, centers)
    return jnp.sum(partials) * (_LAMBDA / (_BATCH * _FEAT))
